# baseline (speedup 1.0000x reference)
import jax
import jax.numpy as jnp
from jax import lax
from jax.experimental import pallas as pl
from jax.experimental.pallas import tpu as pltpu
from jax.experimental.pallas import tpu_sc as plsc

L = 12
SRC = 2048 + 2
TGT = 2048 + 2
D = 1024

_ROWS = L * SRC
_TBL = _ROWS * D

_SC_ROWS = 12312
_TC_ROWS = _ROWS - _SC_ROWS

_SC_E = _SC_ROWS * D
_SC_HALF = _SC_E // 2
_SC_CH = 233472
_SC_NCH = _SC_HALF // _SC_CH
_SC_TOTAL = 3 * _SC_NCH
_SC_NBUF = 6

_TC_RB = 768
_TC_NCH = _TC_ROWS // _TC_RB
_TC_TOTAL = 3 * _TC_NCH
_TC_NBUF = 8


def _sc_body(enc_hbm, self_hbm, cross_hbm, out_hbm, *refs):
    bufs = refs[:_SC_NBUF]
    rsems = refs[_SC_NBUF:2 * _SC_NBUF]
    wsems = refs[2 * _SC_NBUF:]
    cid = lax.axis_index("c")
    sid = lax.axis_index("s")
    base = cid * _SC_HALF
    srcs = (enc_hbm, self_hbm, cross_hbm)

    def rd(k):
        t, c = divmod(k, _SC_NCH)
        b = k % _SC_NBUF
        src = srcs[t].at[pl.ds(base + c * _SC_CH, _SC_CH)]
        return pltpu.make_async_copy(src, bufs[b], rsems[b])

    def wr(k):
        t, c = divmod(k, _SC_NCH)
        b = k % _SC_NBUF
        dst = out_hbm.at[pl.ds(t * _TBL + base + c * _SC_CH, _SC_CH)]
        return pltpu.make_async_copy(bufs[b], dst, wsems[b])

    @pl.when(sid == 0)
    def _():
        rd(0).start()
        for k in range(_SC_TOTAL):
            if k + 1 < _SC_TOTAL:
                if k + 1 >= _SC_NBUF:
                    wr(k + 1 - _SC_NBUF).wait()
                rd(k + 1).start()
            rd(k).wait()
            wr(k).start()
        for j in range(_SC_TOTAL - _SC_NBUF, _SC_TOTAL):
            wr(j).wait()


def _tc_body(enc, selfw, cross, partial, out, buf, rsem, wsem):
    del partial
    srcs = (enc, selfw, cross)

    def rd(k):
        t, c = divmod(k, _TC_NCH)
        b = k % _TC_NBUF
        src = srcs[t].at[pl.ds(_SC_ROWS + c * _TC_RB, _TC_RB), :]
        return pltpu.make_async_copy(src, buf.at[b], rsem.at[b])

    def wr(k):
        t, c = divmod(k, _TC_NCH)
        b = k % _TC_NBUF
        dst = out.at[pl.ds(t * _ROWS + _SC_ROWS + c * _TC_RB, _TC_RB), :]
        return pltpu.make_async_copy(buf.at[b], dst, wsem.at[b])

    rd(0).start()
    for k in range(_TC_TOTAL):
        if k + 1 < _TC_TOTAL:
            if k + 1 >= _TC_NBUF:
                wr(k + 1 - _TC_NBUF).wait()
            rd(k + 1).start()
        rd(k).wait()
        wr(k).start()
    for j in range(_TC_TOTAL - _TC_NBUF, _TC_TOTAL):
        wr(j).wait()


def kernel(bsz, enc_w, self_w, cross_w):
    del bsz
    enc1 = enc_w.reshape(_TBL)
    self1 = self_w.reshape(_TBL)
    cross1 = cross_w.reshape(_TBL)
    mesh = plsc.VectorSubcoreMesh(core_axis_name="c", subcore_axis_name="s")
    sc_run = pl.kernel(
        _sc_body,
        out_type=jax.ShapeDtypeStruct((3 * _TBL,), jnp.float32),
        mesh=mesh,
        scratch_types=(
            [pltpu.VMEM_SHARED((_SC_CH,), jnp.float32)] * _SC_NBUF
            + [pltpu.SemaphoreType.DMA] * (2 * _SC_NBUF)
        ),
    )
    partial = sc_run(enc1, self1, cross1).reshape(3 * _ROWS, D)

    enc2 = enc_w.reshape(_ROWS, D)
    self2 = self_w.reshape(_ROWS, D)
    cross2 = cross_w.reshape(_ROWS, D)
    out = pl.pallas_call(
        _tc_body,
        in_specs=[pl.BlockSpec(memory_space=pl.ANY)] * 4,
        out_specs=pl.BlockSpec(memory_space=pl.ANY),
        out_shape=jax.ShapeDtypeStruct((3 * _ROWS, D), jnp.float32),
        input_output_aliases={3: 0},
        scratch_shapes=[
            pltpu.VMEM((_TC_NBUF, _TC_RB, D), jnp.float32),
            pltpu.SemaphoreType.DMA((_TC_NBUF,)),
            pltpu.SemaphoreType.DMA((_TC_NBUF,)),
        ],
    )(enc2, self2, cross2, partial)
    return out.reshape(3, L, SRC, D)


# speedup vs baseline: 1.8733x; 1.8733x over previous
import jax
import jax.numpy as jnp
from jax.experimental import pallas as pl

L = 12
SRC = 2048 + 2
D = 1024

_ROWS = L * SRC
_RB = 984


def _body(enc, selfw, cross, out):
    out[0] = enc[...]
    out[1] = selfw[...]
    out[2] = cross[...]


def kernel(bsz, enc_w, self_w, cross_w):
    del bsz
    enc2 = enc_w.reshape(_ROWS, D)
    self2 = self_w.reshape(_ROWS, D)
    cross2 = cross_w.reshape(_ROWS, D)
    out = pl.pallas_call(
        _body,
        grid=(_ROWS // _RB,),
        in_specs=[pl.BlockSpec((_RB, D), lambda i: (i, 0))] * 3,
        out_specs=pl.BlockSpec((3, _RB, D), lambda i: (0, i, 0)),
        out_shape=jax.ShapeDtypeStruct((3, _ROWS, D), jnp.float32),
    )(enc2, self2, cross2)
    return out.reshape(3, L, SRC, D)
